# Initial kernel scaffold; baseline (speedup 1.0000x reference)
import jax
import jax.numpy as jnp
from jax.experimental import pallas as pl


def kernel(actions):
    raise NotImplementedError("write your pallas kernel here")


# speedup vs baseline: 17.1396x; 17.1396x over previous
import jax
import jax.numpy as jnp
from jax import lax
from jax.experimental import pallas as pl

NUM_FRAMES = 1024
NUM_TYPES = 4
NUM_ACTIONS = 128
TOTAL_CLS = 256
FB = 8


def _onehot_body(a_ref, o_ref):
    a = a_ref[...]
    col = lax.broadcasted_iota(jnp.int32, (FB, NUM_ACTIONS, TOTAL_CLS), 2)
    tid = col >> 6
    low = col & 63
    res = jnp.zeros((FB, NUM_ACTIONS, TOTAL_CLS), jnp.float32)
    for t in range(NUM_TYPES):
        hit = jnp.logical_and(tid == t, a[:, t, :, None] == low)
        res = jnp.where(hit, 1.0, res)
    o_ref[...] = res


def kernel(actions):
    grid = (NUM_FRAMES // FB,)
    return pl.pallas_call(
        _onehot_body,
        grid=grid,
        in_specs=[
            pl.BlockSpec((FB, NUM_TYPES, NUM_ACTIONS), lambda i: (i, 0, 0))
        ],
        out_specs=pl.BlockSpec(
            (FB, NUM_ACTIONS, TOTAL_CLS), lambda i: (i, 0, 0)
        ),
        out_shape=jax.ShapeDtypeStruct(
            (NUM_FRAMES, NUM_ACTIONS, TOTAL_CLS), jnp.float32
        ),
    )(actions)


# speedup vs baseline: 20.9119x; 1.2201x over previous
import jax
import jax.numpy as jnp
from jax import lax
from jax.experimental import pallas as pl

NUM_FRAMES = 1024
NUM_TYPES = 4
NUM_ACTIONS = 128
TOTAL_CLS = 256
FB = 8


def _onehot_body(a_ref, o_ref):
    a = a_ref[...]
    iota64 = lax.broadcasted_iota(jnp.int32, (FB, NUM_ACTIONS, 64), 2)
    for t in range(NUM_TYPES):
        hit = a[:, t, :, None] == iota64
        o_ref[:, :, t * 64 : (t + 1) * 64] = hit.astype(jnp.float32)


def kernel(actions):
    grid = (NUM_FRAMES // FB,)
    return pl.pallas_call(
        _onehot_body,
        grid=grid,
        in_specs=[
            pl.BlockSpec((FB, NUM_TYPES, NUM_ACTIONS), lambda i: (i, 0, 0))
        ],
        out_specs=pl.BlockSpec(
            (FB, NUM_ACTIONS, TOTAL_CLS), lambda i: (i, 0, 0)
        ),
        out_shape=jax.ShapeDtypeStruct(
            (NUM_FRAMES, NUM_ACTIONS, TOTAL_CLS), jnp.float32
        ),
    )(actions)


# speedup vs baseline: 26.3456x; 1.2598x over previous
import jax
import jax.numpy as jnp
from jax import lax
from jax.experimental import pallas as pl

NUM_FRAMES = 1024
NUM_TYPES = 4
NUM_ACTIONS = 128
TOTAL_CLS = 256
FB = 16


def _onehot_body(a_ref, o_ref):
    a = a_ref[...]
    iota64 = lax.broadcasted_iota(jnp.int32, (FB, NUM_ACTIONS, 64), 2)
    for t in range(NUM_TYPES):
        hit = a[:, t, :, None] == iota64
        o_ref[:, :, t * 64 : (t + 1) * 64] = hit.astype(jnp.float32)


def kernel(actions):
    grid = (NUM_FRAMES // FB,)
    return pl.pallas_call(
        _onehot_body,
        grid=grid,
        in_specs=[
            pl.BlockSpec((FB, NUM_TYPES, NUM_ACTIONS), lambda i: (i, 0, 0))
        ],
        out_specs=pl.BlockSpec(
            (FB, NUM_ACTIONS, TOTAL_CLS), lambda i: (i, 0, 0)
        ),
        out_shape=jax.ShapeDtypeStruct(
            (NUM_FRAMES, NUM_ACTIONS, TOTAL_CLS), jnp.float32
        ),
    )(actions)


# speedup vs baseline: 26.8854x; 1.0205x over previous
import jax
import jax.numpy as jnp
from jax import lax
from jax.experimental import pallas as pl

NUM_FRAMES = 1024
NUM_TYPES = 4
NUM_ACTIONS = 128
TOTAL_CLS = 256
FB = 32


def _onehot_body(a_ref, o_ref):
    a = a_ref[...]
    iota64 = lax.broadcasted_iota(jnp.int32, (FB, NUM_ACTIONS, 64), 2)
    for t in range(NUM_TYPES):
        hit = a[:, t, :, None] == iota64
        o_ref[:, :, t * 64 : (t + 1) * 64] = hit.astype(jnp.float32)


def kernel(actions):
    grid = (NUM_FRAMES // FB,)
    return pl.pallas_call(
        _onehot_body,
        grid=grid,
        in_specs=[
            pl.BlockSpec((FB, NUM_TYPES, NUM_ACTIONS), lambda i: (i, 0, 0))
        ],
        out_specs=pl.BlockSpec(
            (FB, NUM_ACTIONS, TOTAL_CLS), lambda i: (i, 0, 0)
        ),
        out_shape=jax.ShapeDtypeStruct(
            (NUM_FRAMES, NUM_ACTIONS, TOTAL_CLS), jnp.float32
        ),
    )(actions)


# speedup vs baseline: 51.2818x; 1.9074x over previous
import jax
import jax.numpy as jnp
from jax import lax
from jax.experimental import pallas as pl

NUM_FRAMES = 1024
NUM_TYPES = 4
NUM_ACTIONS = 128
TOTAL_CLS = 256
FB = 32


def _onehot_body(a_ref, o_ref):
    a = a_ref[...]
    toff = lax.broadcasted_iota(jnp.int32, (FB, NUM_TYPES, NUM_ACTIONS), 1)
    a2 = (a + (toff << 6)).astype(jnp.float32)
    t_io = lax.broadcasted_iota(jnp.int32, (NUM_TYPES, TOTAL_CLS), 0)
    c_io = lax.broadcasted_iota(jnp.int32, (NUM_TYPES, TOTAL_CLS), 1)
    p = (t_io == (c_io >> 6)).astype(jnp.float32)
    k = lax.dot_general(
        a2, p, (((1,), (0,)), ((), ())), preferred_element_type=jnp.float32
    )
    col = lax.broadcasted_iota(
        jnp.int32, (FB, NUM_ACTIONS, TOTAL_CLS), 2
    ).astype(jnp.float32)
    o_ref[...] = (k == col).astype(jnp.float32)


def kernel(actions):
    grid = (NUM_FRAMES // FB,)
    return pl.pallas_call(
        _onehot_body,
        grid=grid,
        in_specs=[
            pl.BlockSpec((FB, NUM_TYPES, NUM_ACTIONS), lambda i: (i, 0, 0))
        ],
        out_specs=pl.BlockSpec(
            (FB, NUM_ACTIONS, TOTAL_CLS), lambda i: (i, 0, 0)
        ),
        out_shape=jax.ShapeDtypeStruct(
            (NUM_FRAMES, NUM_ACTIONS, TOTAL_CLS), jnp.float32
        ),
    )(actions)


# speedup vs baseline: 56.8244x; 1.1081x over previous
import jax
import jax.numpy as jnp
from jax import lax
from jax.experimental import pallas as pl

NUM_FRAMES = 1024
NUM_TYPES = 4
NUM_ACTIONS = 128
TOTAL_CLS = 256
FB = 64


def _onehot_body(a_ref, o_ref):
    a = a_ref[...]
    toff = lax.broadcasted_iota(jnp.int32, (FB, NUM_TYPES, NUM_ACTIONS), 1)
    a2 = (a + (toff << 6)).astype(jnp.float32)
    t_io = lax.broadcasted_iota(jnp.int32, (NUM_TYPES, TOTAL_CLS), 0)
    c_io = lax.broadcasted_iota(jnp.int32, (NUM_TYPES, TOTAL_CLS), 1)
    p = (t_io == (c_io >> 6)).astype(jnp.float32)
    k = lax.dot_general(
        a2, p, (((1,), (0,)), ((), ())), preferred_element_type=jnp.float32
    )
    col = lax.broadcasted_iota(
        jnp.int32, (FB, NUM_ACTIONS, TOTAL_CLS), 2
    ).astype(jnp.float32)
    o_ref[...] = (k == col).astype(jnp.float32)


def kernel(actions):
    grid = (NUM_FRAMES // FB,)
    return pl.pallas_call(
        _onehot_body,
        grid=grid,
        in_specs=[
            pl.BlockSpec((FB, NUM_TYPES, NUM_ACTIONS), lambda i: (i, 0, 0))
        ],
        out_specs=pl.BlockSpec(
            (FB, NUM_ACTIONS, TOTAL_CLS), lambda i: (i, 0, 0)
        ),
        out_shape=jax.ShapeDtypeStruct(
            (NUM_FRAMES, NUM_ACTIONS, TOTAL_CLS), jnp.float32
        ),
    )(actions)
